# baseline (speedup 1.0000x reference)
import functools

import jax
import jax.numpy as jnp
from jax import lax
from jax.experimental import pallas as pl
from jax.experimental.pallas import tpu as pltpu
from jax.experimental.pallas import tpu_sc as plsc

HIDDEN = 768
E = 16
DFF = 3072
T = 2048
TM = 256
NT = T // TM + E - 1
NC = 2
NS = 16
NW = NC * NS
RPW = T // NW


def _router_plan_kernel(x_ref, w_ref, b_ref, pos_ref, te_ref, used_ref):
    x = x_ref[...]
    logits = jnp.dot(x, w_ref[...], preferred_element_type=jnp.float32) + b_ref[...]
    m = jnp.max(logits, axis=1, keepdims=True)
    is_max = (logits == m).astype(jnp.float32)
    su = (lax.broadcasted_iota(jnp.int32, (E, E), 0)
          < lax.broadcasted_iota(jnp.int32, (E, E), 1)).astype(jnp.float32)
    earlier = jnp.dot(is_max, su, preferred_element_type=jnp.float32)
    sel = is_max * (earlier == 0.0).astype(jnp.float32)
    CH = 128
    NCH = T // CH
    ltc = (lax.broadcasted_iota(jnp.int32, (CH, CH), 0)
           >= lax.broadcasted_iota(jnp.int32, (CH, CH), 1)).astype(jnp.bfloat16)
    sel_b = sel.astype(jnp.bfloat16)
    g = (lax.broadcasted_iota(jnp.int32, (NCH, T), 1) // CH
         == lax.broadcasted_iota(jnp.int32, (NCH, T), 0)).astype(jnp.bfloat16)
    csum = jnp.dot(g, sel_b, preferred_element_type=jnp.float32)
    lcs = (lax.broadcasted_iota(jnp.int32, (NCH, NCH), 0)
           > lax.broadcasted_iota(jnp.int32, (NCH, NCH), 1)).astype(jnp.float32)
    cexcl = jnp.dot(lcs, csum, preferred_element_type=jnp.float32)
    chunks = [
        jnp.dot(ltc, sel_b[c * CH:(c + 1) * CH], preferred_element_type=jnp.float32)
        + cexcl[c:c + 1]
        for c in range(NCH)
    ]
    ranks = jnp.concatenate(chunks, axis=0)
    counts = jnp.sum(sel, axis=0, keepdims=True)
    tiles = jnp.ceil(counts * (1.0 / TM))
    start = jnp.dot(tiles, su, preferred_element_type=jnp.float32)
    row_base = TM * start
    pos_f = jnp.sum(sel * (row_base + ranks - 1.0), axis=1)
    pos_ref[...] = pos_f.astype(jnp.int32)
    used_f = jnp.sum(tiles)
    e_iota = lax.broadcasted_iota(jnp.int32, (1, E), 1).astype(jnp.float32)
    last_e = jnp.max(jnp.where(counts > 0, e_iota, -1.0))
    ii = lax.broadcasted_iota(jnp.int32, (NT, E), 0).astype(jnp.float32)
    cover = ((ii >= start) & (ii < start + tiles)).astype(jnp.float32)
    teg = jnp.sum(cover * e_iota, axis=1)
    i_col = lax.broadcasted_iota(jnp.int32, (NT, 1), 0).astype(jnp.float32)[:, 0]
    te_ref[...] = jnp.where(i_col < used_f, teg, last_e).astype(jnp.int32)
    used_ref[...] = (jnp.zeros((1,), jnp.float32) + used_f).astype(jnp.int32)


def _ffn_kernel(te_ref, used_ref, xs_ref, w1_ref, b1_ref, w2_ref, b2_ref, ys_ref):
    i = pl.program_id(0)

    @pl.when(i < used_ref[0])
    def _():
        h = jnp.dot(xs_ref[...], w1_ref[0], preferred_element_type=jnp.float32)
        h = h + b1_ref[0]
        h = 0.5 * h * (1.0 + lax.erf(h * 0.7071067811865476))
        ys_ref[...] = (
            jnp.dot(h, w2_ref[0], preferred_element_type=jnp.float32) + b2_ref[0]
        )


@functools.lru_cache(maxsize=None)
def _sc_kernels():
    mesh = plsc.VectorSubcoreMesh(
        core_axis_name="c", subcore_axis_name="s", num_cores=NC
    )
    HR = RPW // 2
    scratch = [
        pltpu.VMEM((HR,), jnp.int32),
        pltpu.VMEM((HR,), jnp.int32),
        pltpu.VMEM((HR, HIDDEN), jnp.float32),
        pltpu.VMEM((HR, HIDDEN), jnp.float32),
        pltpu.SemaphoreType.DMA,
        pltpu.SemaphoreType.DMA,
    ]

    @functools.partial(
        pl.kernel,
        mesh=mesh,
        out_type=jax.ShapeDtypeStruct((NT * TM, HIDDEN), jnp.float32),
        scratch_types=scratch,
    )
    def dispatch(x_hbm, pos_hbm, xs_hbm, idx0, idx1, rows0, rows1, sem0, sem1):
        wid = lax.axis_index("s") * NC + lax.axis_index("c")
        base = wid * RPW
        pltpu.sync_copy(pos_hbm.at[pl.ds(base, HR)], idx0)
        a0 = pltpu.async_copy(x_hbm.at[pl.ds(base, HR)], rows0, sem0)
        pltpu.sync_copy(pos_hbm.at[pl.ds(base + HR, HR)], idx1)
        a1 = pltpu.async_copy(x_hbm.at[pl.ds(base + HR, HR)], rows1, sem1)
        a0.wait()
        s0 = pltpu.async_copy(rows0, xs_hbm.at[idx0], sem0)
        a1.wait()
        s1 = pltpu.async_copy(rows1, xs_hbm.at[idx1], sem1)
        s0.wait()
        s1.wait()

    @functools.partial(
        pl.kernel,
        mesh=mesh,
        out_type=jax.ShapeDtypeStruct((T, HIDDEN), jnp.float32),
        scratch_types=scratch,
    )
    def combine(ys_hbm, pos_hbm, out_hbm, idx0, idx1, rows0, rows1, sem0, sem1):
        wid = lax.axis_index("s") * NC + lax.axis_index("c")
        base = wid * RPW
        pltpu.sync_copy(pos_hbm.at[pl.ds(base, HR)], idx0)
        g0 = pltpu.async_copy(ys_hbm.at[idx0], rows0, sem0)
        pltpu.sync_copy(pos_hbm.at[pl.ds(base + HR, HR)], idx1)
        g1 = pltpu.async_copy(ys_hbm.at[idx1], rows1, sem1)
        g0.wait()
        s0 = pltpu.async_copy(rows0, out_hbm.at[pl.ds(base, HR)], sem0)
        g1.wait()
        s1 = pltpu.async_copy(rows1, out_hbm.at[pl.ds(base + HR, HR)], sem1)
        s0.wait()
        s1.wait()

    return dispatch, combine


def _ffn_call(te_s, used_s, xs, W1, b1, W2, b2):
    grid_spec = pltpu.PrefetchScalarGridSpec(
        num_scalar_prefetch=2,
        grid=(NT,),
        in_specs=[
            pl.BlockSpec(
                (TM, HIDDEN),
                lambda i, te, used: (jnp.minimum(i, used[0] - 1), 0),
            ),
            pl.BlockSpec((1, HIDDEN, DFF), lambda i, te, used: (te[i], 0, 0)),
            pl.BlockSpec((1, 1, DFF), lambda i, te, used: (te[i], 0, 0)),
            pl.BlockSpec((1, DFF, HIDDEN), lambda i, te, used: (te[i], 0, 0)),
            pl.BlockSpec((1, 1, HIDDEN), lambda i, te, used: (te[i], 0, 0)),
        ],
        out_specs=pl.BlockSpec(
            (TM, HIDDEN),
            lambda i, te, used: (jnp.minimum(i, used[0] - 1), 0),
        ),
    )
    return pl.pallas_call(
        _ffn_kernel,
        grid_spec=grid_spec,
        out_shape=jax.ShapeDtypeStruct((NT * TM, HIDDEN), jnp.float32),
    )(te_s, used_s, xs, W1, b1, W2, b2)


@jax.jit
def kernel(x, router_w, router_b, W1, b1, W2, b2):
    B, S, D = x.shape
    x2d = x.reshape(T, D)
    pos, te, used = pl.pallas_call(
        _router_plan_kernel,
        out_shape=[
            jax.ShapeDtypeStruct((T,), jnp.int32),
            jax.ShapeDtypeStruct((NT,), jnp.int32),
            jax.ShapeDtypeStruct((1,), jnp.int32),
        ],
    )(x2d, router_w, router_b.reshape(1, E))
    dispatch, combine = _sc_kernels()
    xs = dispatch(x2d, pos)
    ys = _ffn_call(
        te, used, xs,
        W1, b1.reshape(E, 1, DFF), W2, b2.reshape(E, 1, HIDDEN),
    )
    out = combine(ys, pos)
    return out.reshape(B, S, D)


# speedup vs baseline: 1.0072x; 1.0072x over previous
import functools

import jax
import jax.numpy as jnp
from jax import lax
from jax.experimental import pallas as pl
from jax.experimental.pallas import tpu as pltpu
from jax.experimental.pallas import tpu_sc as plsc

HIDDEN = 768
E = 16
DFF = 3072
T = 2048
TM = 256
NT = T // TM + E - 1
NC = 2
NS = 16
NW = NC * NS
RPW = T // NW


def _router_plan_kernel(x_ref, w_ref, b_ref, pos_ref, te_ref, used_ref):
    x = x_ref[...]
    logits = jnp.dot(x, w_ref[...], preferred_element_type=jnp.float32) + b_ref[...]
    m = jnp.max(logits, axis=1, keepdims=True)
    is_max = (logits == m).astype(jnp.float32)
    su = (lax.broadcasted_iota(jnp.int32, (E, E), 0)
          < lax.broadcasted_iota(jnp.int32, (E, E), 1)).astype(jnp.float32)
    earlier = jnp.dot(is_max, su, preferred_element_type=jnp.float32)
    sel = is_max * (earlier == 0.0).astype(jnp.float32)
    CH = 128
    NCH = T // CH
    ltc = (lax.broadcasted_iota(jnp.int32, (CH, CH), 0)
           >= lax.broadcasted_iota(jnp.int32, (CH, CH), 1)).astype(jnp.bfloat16)
    sel_b = sel.astype(jnp.bfloat16)
    g = (lax.broadcasted_iota(jnp.int32, (NCH, T), 1) // CH
         == lax.broadcasted_iota(jnp.int32, (NCH, T), 0)).astype(jnp.bfloat16)
    csum = jnp.dot(g, sel_b, preferred_element_type=jnp.float32)
    lcs = (lax.broadcasted_iota(jnp.int32, (NCH, NCH), 0)
           > lax.broadcasted_iota(jnp.int32, (NCH, NCH), 1)).astype(jnp.float32)
    cexcl = jnp.dot(lcs, csum, preferred_element_type=jnp.float32)
    chunks = [
        jnp.dot(ltc, sel_b[c * CH:(c + 1) * CH], preferred_element_type=jnp.float32)
        + cexcl[c:c + 1]
        for c in range(NCH)
    ]
    ranks = jnp.concatenate(chunks, axis=0)
    counts = jnp.sum(sel, axis=0, keepdims=True)
    tiles = jnp.ceil(counts * (1.0 / TM))
    start = jnp.dot(tiles, su, preferred_element_type=jnp.float32)
    row_base = TM * start
    pos_f = jnp.sum(sel * (row_base + ranks - 1.0), axis=1)
    pos_ref[...] = pos_f.astype(jnp.int32)
    used_f = jnp.sum(tiles)
    e_iota = lax.broadcasted_iota(jnp.int32, (1, E), 1).astype(jnp.float32)
    last_e = jnp.max(jnp.where(counts > 0, e_iota, -1.0))
    ii = lax.broadcasted_iota(jnp.int32, (NT, E), 0).astype(jnp.float32)
    cover = ((ii >= start) & (ii < start + tiles)).astype(jnp.float32)
    teg = jnp.sum(cover * e_iota, axis=1)
    i_col = lax.broadcasted_iota(jnp.int32, (NT, 1), 0).astype(jnp.float32)[:, 0]
    te_ref[...] = jnp.where(i_col < used_f, teg, last_e).astype(jnp.int32)
    used_ref[...] = (jnp.zeros((1,), jnp.float32) + used_f).astype(jnp.int32)


def _ffn_kernel(te_ref, used_ref, xs_ref, w1_ref, b1_ref, w2_ref, b2_ref, ys_ref):
    i = pl.program_id(0)

    @pl.when(i < used_ref[0])
    def _():
        h = jnp.dot(xs_ref[...], w1_ref[0], preferred_element_type=jnp.float32)
        h = h + b1_ref[0]
        h = 0.5 * h * (1.0 + lax.erf(h * 0.7071067811865476))
        ys_ref[...] = (
            jnp.dot(h, w2_ref[0], preferred_element_type=jnp.float32) + b2_ref[0]
        )


@functools.lru_cache(maxsize=None)
def _sc_kernels():
    mesh = plsc.VectorSubcoreMesh(
        core_axis_name="c", subcore_axis_name="s", num_cores=NC
    )
    scratch = [
        pltpu.VMEM((RPW,), jnp.int32),
        pltpu.VMEM((RPW, HIDDEN), jnp.float32),
        pltpu.SemaphoreType.DMA,
    ]

    @functools.partial(
        pl.kernel,
        mesh=mesh,
        out_type=jax.ShapeDtypeStruct((NT * TM, HIDDEN), jnp.float32),
        scratch_types=scratch,
    )
    def dispatch(x_hbm, pos_hbm, xs_hbm, idx_v, rows_v, sem):
        wid = lax.axis_index("s") * NC + lax.axis_index("c")
        base = wid * RPW
        pltpu.sync_copy(pos_hbm.at[pl.ds(base, RPW)], idx_v)
        pltpu.sync_copy(x_hbm.at[pl.ds(base, RPW)], rows_v)
        pltpu.async_copy(rows_v, xs_hbm.at[idx_v], sem).wait()

    @functools.partial(
        pl.kernel,
        mesh=mesh,
        out_type=jax.ShapeDtypeStruct((T, HIDDEN), jnp.float32),
        scratch_types=scratch,
    )
    def combine(ys_hbm, pos_hbm, out_hbm, idx_v, rows_v, sem):
        wid = lax.axis_index("s") * NC + lax.axis_index("c")
        base = wid * RPW
        pltpu.sync_copy(pos_hbm.at[pl.ds(base, RPW)], idx_v)
        pltpu.async_copy(ys_hbm.at[idx_v], rows_v, sem).wait()
        pltpu.sync_copy(rows_v, out_hbm.at[pl.ds(base, RPW)])

    return dispatch, combine


def _ffn_call(te_s, used_s, xs, W1, b1, W2, b2):
    grid_spec = pltpu.PrefetchScalarGridSpec(
        num_scalar_prefetch=2,
        grid=(NT,),
        in_specs=[
            pl.BlockSpec(
                (TM, HIDDEN),
                lambda i, te, used: (jnp.minimum(i, used[0] - 1), 0),
            ),
            pl.BlockSpec((1, HIDDEN, DFF), lambda i, te, used: (te[i], 0, 0)),
            pl.BlockSpec((1, 1, DFF), lambda i, te, used: (te[i], 0, 0)),
            pl.BlockSpec((1, DFF, HIDDEN), lambda i, te, used: (te[i], 0, 0)),
            pl.BlockSpec((1, 1, HIDDEN), lambda i, te, used: (te[i], 0, 0)),
        ],
        out_specs=pl.BlockSpec(
            (TM, HIDDEN),
            lambda i, te, used: (jnp.minimum(i, used[0] - 1), 0),
        ),
    )
    return pl.pallas_call(
        _ffn_kernel,
        grid_spec=grid_spec,
        out_shape=jax.ShapeDtypeStruct((NT * TM, HIDDEN), jnp.float32),
    )(te_s, used_s, xs, W1, b1, W2, b2)


@jax.jit
def kernel(x, router_w, router_b, W1, b1, W2, b2):
    B, S, D = x.shape
    x2d = x.reshape(T, D)
    pos, te, used = pl.pallas_call(
        _router_plan_kernel,
        out_shape=[
            jax.ShapeDtypeStruct((T,), jnp.int32),
            jax.ShapeDtypeStruct((NT,), jnp.int32),
            jax.ShapeDtypeStruct((1,), jnp.int32),
        ],
    )(x2d, router_w, router_b.reshape(1, E))
    dispatch, combine = _sc_kernels()
    xs = dispatch(x2d, pos)
    ys = _ffn_call(
        te, used, xs,
        W1, b1.reshape(E, 1, DFF), W2, b2.reshape(E, 1, HIDDEN),
    )
    out = combine(ys, pos)
    return out.reshape(B, S, D)
